# Initial kernel scaffold; baseline (speedup 1.0000x reference)
import jax
import jax.numpy as jnp
from jax.experimental import pallas as pl


def kernel(fragments, alphas, ptclds):
    raise NotImplementedError("write your pallas kernel here")


# speedup vs baseline: 48.3247x; 48.3247x over previous
import jax
import jax.numpy as jnp
from jax import lax
from jax.experimental import pallas as pl
from jax.experimental.pallas import tpu as pltpu, tpu_sc as plsc

N, K, H, W = 4, 16, 256, 256
C, P = 4, 100000
HW = H * W
NPIX = N * HW
NWORK = 32
SLOTS = NWORK // C
CHUNK = NPIX // SLOTS
S = 256
NBLK = CHUNK // S
NV = S // 16


def _sc_composite(frag_hbm, alpha_hbm, table_hbm, out_hbm,
                  table_v, frag_v, alpha_v, out_v):
    cid = lax.axis_index("c")
    sid = lax.axis_index("s")
    wid = sid * 2 + cid
    g = wid % C
    slot = wid // C
    pix0 = slot * CHUNK
    n = pix0 // HW
    hw0 = pix0 % HW

    pltpu.sync_copy(table_hbm.at[g], table_v)

    def block_body(b, carry):
        off = hw0 + b * S
        pltpu.sync_copy(frag_hbm.at[n, :, pl.ds(off, S)], frag_v)
        pltpu.sync_copy(alpha_hbm.at[n, :, pl.ds(off, S)], alpha_v)

        def vbody(v, c2):
            base = v * 16
            trans = jnp.full((16,), 1.0, jnp.float32)
            acc = jnp.zeros((16,), jnp.float32)
            for k in range(K):
                f = frag_v[k, pl.ds(base, 16)]
                a = alpha_v[k, pl.ds(base, 16)]
                vals = plsc.load_gather(table_v, [f])
                w = a * trans
                acc = acc + w * vals
                if k + 1 < K:
                    trans = trans * (1.0 - a)
            out_v[pl.ds(base, 16)] = acc
            return c2

        lax.fori_loop(0, NV, vbody, 0)
        pltpu.sync_copy(out_v, out_hbm.at[n, g, pl.ds(off, S)])
        return carry

    lax.fori_loop(0, NBLK, block_body, 0)


@jax.jit
def kernel(fragments, alphas, ptclds):
    frag = fragments.astype(jnp.int32).reshape(N, K, HW)
    al = alphas.reshape(N, K, HW)
    run = pl.kernel(
        _sc_composite,
        out_type=jax.ShapeDtypeStruct((N, C, HW), jnp.float32),
        mesh=plsc.VectorSubcoreMesh(core_axis_name="c", subcore_axis_name="s", num_cores=2, num_subcores=16),
        compiler_params=pltpu.CompilerParams(needs_layout_passes=False),
        scratch_types=[
            pltpu.VMEM((P,), jnp.float32),
            pltpu.VMEM((K, S), jnp.int32),
            pltpu.VMEM((K, S), jnp.float32),
            pltpu.VMEM((S,), jnp.float32),
        ],
    )
    out = run(frag, al, ptclds)
    return out.reshape(N, C, H, W)
